# baseline (speedup 1.0000x reference)
import jax
import jax.numpy as jnp
from jax import lax
from jax.experimental import pallas as pl
from jax.experimental.pallas import tpu as pltpu
from jax.experimental.pallas import tpu_sc as plsc

N = 50000
E = 800000
G = 512
NPAD = 50176
HID = 128
CW = 32
NCHUNK = 4
NCORE = 2
NSUB = 16
NTILE = NCORE * NSUB
EPT = E // NTILE
BLK = 125
NB = EPT // BLK
SEGB = 40
NSEG = NB // SEGB
RING = 4
LAG = 2
ZROWS = 98
SROWS = NPAD // NSUB
IDT = NPAD // NTILE
IDB = 112
IDNB = IDT // IDB
BR = 1024
NBLOCKS = NPAD // BR


def _emb_body(ids_nt, ids_id, ta, tb, tc_, xc0, xc1, xc2,
              idv_nt, idv_id, ba, bb, bc, ga, gb, gc):
    cid = lax.axis_index("c")
    sid = lax.axis_index("s")
    tid = sid * NCORE + cid
    base = tid * IDT
    pltpu.sync_copy(ids_nt.at[tid], idv_nt)
    pltpu.sync_copy(ids_id.at[tid], idv_id)
    pltpu.async_copy(ta.at[idv_nt.at[0]], ba, ga)
    pltpu.async_copy(tb.at[idv_nt.at[0]], bb, gb)
    pltpu.async_copy(tc_.at[idv_id.at[0]], bc, gc)

    def step(k, c):
        for tbl, idv, buf, sem, out in (
            (ta, idv_nt, ba, ga, xc0),
            (tb, idv_nt, bb, gb, xc1),
            (tc_, idv_id, bc, gc, xc2),
        ):
            pltpu.make_async_copy(tbl.at[idv.at[0]], buf, sem).wait()
            pltpu.sync_copy(buf, out.at[pl.ds(base + k * IDB, IDB)])

            @pl.when(k + 1 < IDNB)
            def _():
                pltpu.async_copy(tbl.at[idv.at[k + 1]], buf, sem)
        return c

    lax.fori_loop(0, IDNB, step, 0)


def _agg_body(ed_hbm, t0, t1, agg_out,
              ed_v, b0, b1, b2, b3, zbuf, slab,
              g0, g1, g2, g3, s0, s1, s2, s3):
    cid = lax.axis_index("c")
    sid = lax.axis_index("s")
    tid = sid * NCORE + cid
    bufs = (b0, b1, b2, b3)
    gsem = (g0, g1, g2, g3)
    ssem = (s0, s1, s2, s3)

    def zrow(i, c):
        zbuf[i, pl.ds(0, 16)] = jnp.zeros((16,), jnp.float32)
        zbuf[i, pl.ds(16, 16)] = jnp.zeros((16,), jnp.float32)
        return c

    lax.fori_loop(0, ZROWS, zrow, 0)

    for f, tbl in enumerate((t0, t1)):
        for z in range(SROWS // ZROWS):
            pltpu.sync_copy(zbuf, slab.at[pl.ds(sid * SROWS + z * ZROWS, ZROWS)])
        plsc.subcore_barrier()

        for seg in range(NSEG):
            pltpu.sync_copy(ed_hbm.at[tid, seg], ed_v)

            def gwait(u):
                pltpu.make_async_copy(tbl.at[ed_v.at[0, 0]], bufs[u],
                                      gsem[u]).wait()

            def swait(u):
                pltpu.make_async_copy(bufs[u], slab.at[ed_v.at[1, 0]],
                                      ssem[u]).wait()

            for u in range(RING):
                pltpu.async_copy(tbl.at[ed_v.at[0, u]], bufs[u], gsem[u])

            def blk4(jo, c):
                for u in range(RING):
                    j = jo * RING + u
                    gwait(u)
                    pltpu.sync_copy(bufs[u], slab.at[ed_v.at[1, j]], add=True)

                    @pl.when(j + RING < SEGB)
                    def _():
                        pltpu.async_copy(tbl.at[ed_v.at[0, j + RING]], bufs[u],
                                         gsem[u])
                return c

            lax.fori_loop(0, SEGB // RING, blk4, 0)
        plsc.subcore_barrier()
        pltpu.sync_copy(slab.at[pl.ds(sid * SROWS, SROWS)],
                        agg_out.at[cid, f, pl.ds(sid * SROWS, SROWS)])
        plsc.subcore_barrier()


def _max_body(d_ref, o_ref):
    o_ref[0, 0] = jnp.max(d_ref[...])


def _mm(a, b_t):
    return lax.dot_general(a, b_t, (((1,), (1,)), ((), ())),
                           preferred_element_type=jnp.float32)


def _self_body(c0, c1, c2, c3, w, b, o):
    x = jnp.concatenate([c0[...], c1[...], c2[...], c3[...]], axis=1)
    o[...] = _mm(x, w[...]) + b[...]


def _nbA_body(aggA, wn, o):
    aA = aggA[...]
    sA = aA[0] + aA[1]
    wnv = wn[...]
    o[...] = _mm(sA[0], wnv[:, 0:32]) + _mm(sA[1], wnv[:, 32:64])


def _l0_body(sf, nbA, aggB, wn0, bn0, o0, o1, o2, o3):
    aB = aggB[...]
    sB = aB[0] + aB[1]
    deg = jnp.clip(sB[1][:, 31:32], 1.0)
    wn = wn0[...]
    nb = nbA[...] + _mm(sB[0], wn[:, 64:96]) + _mm(sB[1], wn[:, 96:128])
    x1 = jnp.maximum(sf[...] + nb / deg + bn0[...], 0.0)
    o0[...] = x1[:, 0:32]
    o1[...] = x1[:, 32:64]
    o2[...] = x1[:, 64:96]
    o3[...] = x1[:, 96:128]


def _l1_body(sf, nbA, aggB, a0c3, bi, wn1, bn1, pw, pb, out, pooled):
    i = pl.program_id(0)
    aB = aggB[...]
    sB = aB[0] + aB[1]
    a0 = a0c3[...]
    d0 = a0[0, 0] + a0[1, 0]
    deg = jnp.clip(d0[:, 31:32], 1.0)
    wn = wn1[...]
    nb = nbA[...] + _mm(sB[0], wn[:, 64:96]) + _mm(sB[1], wn[:, 96:128])
    x2 = jnp.maximum(sf[...] + nb / deg + bn1[...], 0.0)
    prj = _mm(x2, pw[...])
    lanes = lax.broadcasted_iota(jnp.int32, (BR, 128), 1)
    prj = prj + (lanes == 14).astype(jnp.float32)
    b = bi[...]

    @pl.when(i == 0)
    def _():
        pooled[...] = jnp.zeros((4 * 128, 128), jnp.float32)

    for gc in range(4):
        oh = (b == lanes + gc * 128).astype(jnp.float32)
        contrib = lax.dot_general(oh, prj, (((0,), (0,)), ((), ())),
                                  preferred_element_type=jnp.float32)
        pooled[pl.ds(gc * 128, 128), :] += contrib

    @pl.when(i == NBLOCKS - 1)
    def _():
        pv = pooled[...]
        cnt = jnp.clip(pv[:, 14:15], 1.0)
        out[...] = pv / cnt + pb[...]


def kernel(node_type_ids, ident_hash_ids, literal_flags, node_depth, edge_index,
           batch_index, risk_labels, category_labels,
           node_type_table, ident_table,
           W_self0, b_self0, W_neigh0, b_neigh0,
           W_self1, b_self1, W_neigh1, b_neigh1,
           risk_W, risk_b, cat_W, cat_b):
    f32 = jnp.float32
    pad = NPAD - N
    mesh = plsc.VectorSubcoreMesh(core_axis_name="c", subcore_axis_name="s")

    nt_ids3 = jnp.pad(node_type_ids.astype(jnp.int32), (0, pad)).reshape(
        NTILE, IDNB, IDB)
    id_ids3 = jnp.pad(ident_hash_ids.astype(jnp.int32), (0, pad)).reshape(
        NTILE, IDNB, IDB)
    nt_a = node_type_table[:, 0:32]
    nt_b = node_type_table[:, 32:64]
    ed4 = jnp.stack([
        edge_index[0].astype(jnp.int32).reshape(NTILE, NSEG, SEGB, BLK),
        edge_index[1].astype(jnp.int32).reshape(NTILE, NSEG, SEGB, BLK),
    ], axis=2)

    depth_f = node_depth.astype(f32)
    depth2 = jnp.pad(depth_f, (0, pad)).reshape(NPAD // 128, 128)
    dmax = pl.pallas_call(
        _max_body,
        out_shape=jax.ShapeDtypeStruct((1, 1), f32),
        out_specs=pl.BlockSpec(memory_space=pltpu.SMEM),
    )(depth2)
    depth_n = depth_f / jnp.clip(dmax[0, 0], 1.0)
    xc3 = jnp.concatenate([
        literal_flags, depth_n[:, None],
        jnp.zeros((N, CW - 7), f32), jnp.ones((N, 1), f32)], axis=1)
    xc3 = jnp.pad(xc3, ((0, pad), (0, 0)))

    w0p = jnp.pad(W_self0, ((0, 0), (0, HID - W_self0.shape[1])))
    wn0p = jnp.pad(W_neigh0, ((0, 0), (0, HID - W_neigh0.shape[1])))
    bs0 = b_self0[None, :]
    bn0 = b_neigh0[None, :]
    bs1 = b_self1[None, :]
    bn1 = b_neigh1[None, :]
    pw = jnp.zeros((HID, HID), f32).at[0:1].set(risk_W).at[1:13].set(cat_W)
    pb = jnp.zeros((1, HID), f32).at[0, 0].set(risk_b[0]).at[0, 1:13].set(cat_b)
    bi_bc = jnp.broadcast_to(
        jnp.pad(batch_index.astype(jnp.int32), (0, pad), constant_values=G)[:, None],
        (NPAD, 128))

    xc0, xc1, xc2 = pl.kernel(
        _emb_body,
        out_type=[jax.ShapeDtypeStruct((NPAD, CW), f32)] * 3,
        mesh=mesh,
        scratch_types=[pltpu.VMEM((IDNB, IDB), jnp.int32)] * 2
        + [pltpu.VMEM((IDB, CW), f32)] * 3
        + [pltpu.SemaphoreType.DMA] * 3,
        compiler_params=pltpu.CompilerParams(use_tc_tiling_on_sc=False),
    )(nt_ids3, id_ids3, nt_a, nt_b, ident_table)

    def sc_aggregate(t0, t1):
        return pl.kernel(
            _agg_body,
            out_type=jax.ShapeDtypeStruct((NCORE, 2, NPAD, CW), f32),
            mesh=mesh,
            scratch_types=[pltpu.VMEM((2, SEGB, BLK), jnp.int32)]
            + [pltpu.VMEM((BLK, CW), f32)] * RING
            + [pltpu.VMEM((ZROWS, CW), f32),
               pltpu.VMEM_SHARED((NPAD, CW), f32)]
            + [pltpu.SemaphoreType.DMA] * (2 * RING),
            compiler_params=pltpu.CompilerParams(use_tc_tiling_on_sc=False),
        )(ed4, t0, t1)

    b32 = pl.BlockSpec((BR, CW), lambda i: (i, 0))
    b128 = pl.BlockSpec((BR, HID), lambda i: (i, 0))
    bagg = pl.BlockSpec((NCORE, 2, BR, CW), lambda i: (0, 0, i, 0))
    bw = pl.BlockSpec((HID, HID), lambda i: (0, 0))
    bb = pl.BlockSpec((1, HID), lambda i: (0, 0))

    def self_mm(chunks, w, b):
        return pl.pallas_call(
            _self_body,
            grid=(NBLOCKS,),
            in_specs=[b32] * 4 + [bw, bb],
            out_specs=b128,
            out_shape=jax.ShapeDtypeStruct((NPAD, HID), f32),
        )(*chunks, w, b)

    def nbA_mm(aggA, wn):
        return pl.pallas_call(
            _nbA_body,
            grid=(NBLOCKS,),
            in_specs=[bagg, bw],
            out_specs=b128,
            out_shape=jax.ShapeDtypeStruct((NPAD, HID), f32),
        )(aggA, wn)

    aggA0 = sc_aggregate(xc0, xc1)
    aggB0 = sc_aggregate(xc2, xc3)
    self0 = self_mm((xc0, xc1, xc2, xc3), w0p, bs0)
    nbA0 = nbA_mm(aggA0, wn0p)
    x1c = pl.pallas_call(
        _l0_body,
        grid=(NBLOCKS,),
        in_specs=[b128, b128, bagg, bw, bb],
        out_specs=[b32] * 4,
        out_shape=[jax.ShapeDtypeStruct((NPAD, CW), f32)] * 4,
    )(self0, nbA0, aggB0, wn0p, bn0)

    aggA1 = sc_aggregate(x1c[0], x1c[1])
    aggB1 = sc_aggregate(x1c[2], x1c[3])
    self1 = self_mm(x1c, W_self1, bs1)
    nbA1 = nbA_mm(aggA1, W_neigh1)

    ba0 = pl.BlockSpec((NCORE, 1, BR, CW), lambda i: (0, 1, i, 0))
    bbi = pl.BlockSpec((BR, 128), lambda i: (i, 0))
    out = pl.pallas_call(
        _l1_body,
        grid=(NBLOCKS,),
        in_specs=[b128, b128, bagg, ba0, bbi, bw, bb, bw, bb],
        out_specs=pl.BlockSpec((4 * 128, HID), lambda i: (0, 0)),
        out_shape=jax.ShapeDtypeStruct((4 * 128, HID), f32),
        scratch_shapes=[pltpu.VMEM((4 * 128, HID), f32)],
    )(self1, nbA1, aggB1, aggB0, bi_bc, W_neigh1, bn1, pw, pb)

    return (out[:G, 0], out[:G, 1:13])


# speedup vs baseline: 1.0592x; 1.0592x over previous
import jax
import jax.numpy as jnp
from jax import lax
from jax.experimental import pallas as pl
from jax.experimental.pallas import tpu as pltpu
from jax.experimental.pallas import tpu_sc as plsc

N = 50000
E = 800000
G = 512
NPAD = 50176
HID = 128
CW = 32
NCHUNK = 4
NCORE = 2
NSUB = 16
NTILE = NCORE * NSUB
EPT = E // NTILE
BLK = 125
NB = EPT // BLK
SEGB = 40
NSEG = NB // SEGB
RING = 4
LAG = 2
ZROWS = 98
SROWS = NPAD // NSUB
IDT = NPAD // NTILE
IDB = 112
IDNB = IDT // IDB
BR = 1024
NBLOCKS = NPAD // BR


def _emb_body(ids_nt, ids_id, ta, tb, tc_, xc0, xc1, xc2,
              idv_nt, idv_id, ba, bb, bc, ga, gb, gc):
    cid = lax.axis_index("c")
    sid = lax.axis_index("s")
    tid = sid * NCORE + cid
    base = tid * IDT
    pltpu.sync_copy(ids_nt.at[tid], idv_nt)
    pltpu.sync_copy(ids_id.at[tid], idv_id)
    pltpu.async_copy(ta.at[idv_nt.at[0]], ba, ga)
    pltpu.async_copy(tb.at[idv_nt.at[0]], bb, gb)
    pltpu.async_copy(tc_.at[idv_id.at[0]], bc, gc)

    def step(k, c):
        for tbl, idv, buf, sem, out in (
            (ta, idv_nt, ba, ga, xc0),
            (tb, idv_nt, bb, gb, xc1),
            (tc_, idv_id, bc, gc, xc2),
        ):
            pltpu.make_async_copy(tbl.at[idv.at[0]], buf, sem).wait()
            pltpu.sync_copy(buf, out.at[pl.ds(base + k * IDB, IDB)])

            @pl.when(k + 1 < IDNB)
            def _():
                pltpu.async_copy(tbl.at[idv.at[k + 1]], buf, sem)
        return c

    lax.fori_loop(0, IDNB, step, 0)


def _agg_body(ed_hbm, t0, t1, agg_out,
              ed_v, b0, b1, b2, b3, zbuf, slab,
              g0, g1, g2, g3, s0, s1, s2, s3):
    cid = lax.axis_index("c")
    sid = lax.axis_index("s")
    tid = sid * NCORE + cid
    bufs = (b0, b1, b2, b3)
    gsem = (g0, g1, g2, g3)
    ssem = (s0, s1, s2, s3)

    def zrow(i, c):
        zbuf[i, pl.ds(0, 16)] = jnp.zeros((16,), jnp.float32)
        zbuf[i, pl.ds(16, 16)] = jnp.zeros((16,), jnp.float32)
        return c

    lax.fori_loop(0, ZROWS, zrow, 0)

    for f, tbl in enumerate((t0, t1)):
        for z in range(SROWS // ZROWS):
            pltpu.sync_copy(zbuf, slab.at[pl.ds(sid * SROWS + z * ZROWS, ZROWS)])
        plsc.subcore_barrier()

        for seg in range(NSEG):
            pltpu.sync_copy(ed_hbm.at[tid, seg], ed_v)

            def gwait(u):
                pltpu.make_async_copy(tbl.at[ed_v.at[0, 0]], bufs[u],
                                      gsem[u]).wait()

            def swait(u):
                pltpu.make_async_copy(bufs[u], slab.at[ed_v.at[1, 0]],
                                      ssem[u]).wait()

            for u in range(RING):
                pltpu.async_copy(tbl.at[ed_v.at[0, u]], bufs[u], gsem[u])

            def blk4(jo, c):
                for u in range(RING):
                    j = jo * RING + u
                    gwait(u)
                    pltpu.sync_copy(bufs[u], slab.at[ed_v.at[1, j]], add=True)

                    @pl.when(j + RING < SEGB)
                    def _():
                        pltpu.async_copy(tbl.at[ed_v.at[0, j + RING]], bufs[u],
                                         gsem[u])
                return c

            lax.fori_loop(0, SEGB // RING, blk4, 0)
        plsc.subcore_barrier()
        pltpu.sync_copy(slab.at[pl.ds(sid * SROWS, SROWS)],
                        agg_out.at[cid, f, pl.ds(sid * SROWS, SROWS)])
        plsc.subcore_barrier()


def _max_body(d_ref, o_ref):
    o_ref[0, 0] = jnp.max(d_ref[...])


def _mm(a, b_t):
    return lax.dot_general(a, b_t, (((1,), (1,)), ((), ())),
                           preferred_element_type=jnp.float32)


def _self_body(c0, c1, c2, c3, w, b, o):
    x = jnp.concatenate([c0[...], c1[...], c2[...], c3[...]], axis=1)
    o[...] = _mm(x, w[...]) + b[...]


def _l0_body(sf, aggA, aggB, wn0, bn0, o0, o1, o2, o3):
    aA = aggA[...]
    aB = aggB[...]
    sA = aA[0] + aA[1]
    sB = aB[0] + aB[1]
    deg = jnp.clip(sB[1][:, 31:32], 1.0)
    wn = wn0[...]
    nb = (_mm(sA[0], wn[:, 0:32]) + _mm(sA[1], wn[:, 32:64])
          + _mm(sB[0], wn[:, 64:96]) + _mm(sB[1], wn[:, 96:128]))
    x1 = jnp.maximum(sf[...] + nb / deg + bn0[...], 0.0)
    o0[...] = x1[:, 0:32]
    o1[...] = x1[:, 32:64]
    o2[...] = x1[:, 64:96]
    o3[...] = x1[:, 96:128]


def _l1_body(sf, aggA, aggB, a0c3, bi, wn1, bn1, pw, pb, out, pooled):
    i = pl.program_id(0)
    aA = aggA[...]
    aB = aggB[...]
    sA = aA[0] + aA[1]
    sB = aB[0] + aB[1]
    a0 = a0c3[...]
    d0 = a0[0, 0] + a0[1, 0]
    deg = jnp.clip(d0[:, 31:32], 1.0)
    wn = wn1[...]
    nb = (_mm(sA[0], wn[:, 0:32]) + _mm(sA[1], wn[:, 32:64])
          + _mm(sB[0], wn[:, 64:96]) + _mm(sB[1], wn[:, 96:128]))
    x2 = jnp.maximum(sf[...] + nb / deg + bn1[...], 0.0)
    prj = _mm(x2, pw[...])
    lanes = lax.broadcasted_iota(jnp.int32, (BR, 128), 1)
    prj = prj + (lanes == 14).astype(jnp.float32)
    b = bi[...]

    @pl.when(i == 0)
    def _():
        pooled[...] = jnp.zeros((4 * 128, 128), jnp.float32)

    for gc in range(4):
        oh = (b == lanes + gc * 128).astype(jnp.float32)
        contrib = lax.dot_general(oh, prj, (((0,), (0,)), ((), ())),
                                  preferred_element_type=jnp.float32)
        pooled[pl.ds(gc * 128, 128), :] += contrib

    @pl.when(i == NBLOCKS - 1)
    def _():
        pv = pooled[...]
        cnt = jnp.clip(pv[:, 14:15], 1.0)
        out[...] = pv / cnt + pb[...]


def kernel(node_type_ids, ident_hash_ids, literal_flags, node_depth, edge_index,
           batch_index, risk_labels, category_labels,
           node_type_table, ident_table,
           W_self0, b_self0, W_neigh0, b_neigh0,
           W_self1, b_self1, W_neigh1, b_neigh1,
           risk_W, risk_b, cat_W, cat_b):
    f32 = jnp.float32
    pad = NPAD - N
    mesh = plsc.VectorSubcoreMesh(core_axis_name="c", subcore_axis_name="s")

    nt_ids3 = jnp.pad(node_type_ids.astype(jnp.int32), (0, pad)).reshape(
        NTILE, IDNB, IDB)
    id_ids3 = jnp.pad(ident_hash_ids.astype(jnp.int32), (0, pad)).reshape(
        NTILE, IDNB, IDB)
    nt_a = node_type_table[:, 0:32]
    nt_b = node_type_table[:, 32:64]
    ed4 = jnp.stack([
        edge_index[0].astype(jnp.int32).reshape(NTILE, NSEG, SEGB, BLK),
        edge_index[1].astype(jnp.int32).reshape(NTILE, NSEG, SEGB, BLK),
    ], axis=2)

    depth_f = node_depth.astype(f32)
    depth2 = jnp.pad(depth_f, (0, pad)).reshape(NPAD // 128, 128)
    dmax = pl.pallas_call(
        _max_body,
        out_shape=jax.ShapeDtypeStruct((1, 1), f32),
        out_specs=pl.BlockSpec(memory_space=pltpu.SMEM),
    )(depth2)
    depth_n = depth_f / jnp.clip(dmax[0, 0], 1.0)
    xc3 = jnp.concatenate([
        literal_flags, depth_n[:, None],
        jnp.zeros((N, CW - 7), f32), jnp.ones((N, 1), f32)], axis=1)
    xc3 = jnp.pad(xc3, ((0, pad), (0, 0)))

    w0p = jnp.pad(W_self0, ((0, 0), (0, HID - W_self0.shape[1])))
    wn0p = jnp.pad(W_neigh0, ((0, 0), (0, HID - W_neigh0.shape[1])))
    bs0 = b_self0[None, :]
    bn0 = b_neigh0[None, :]
    bs1 = b_self1[None, :]
    bn1 = b_neigh1[None, :]
    pw = jnp.zeros((HID, HID), f32).at[0:1].set(risk_W).at[1:13].set(cat_W)
    pb = jnp.zeros((1, HID), f32).at[0, 0].set(risk_b[0]).at[0, 1:13].set(cat_b)
    bi_bc = jnp.broadcast_to(
        jnp.pad(batch_index.astype(jnp.int32), (0, pad), constant_values=G)[:, None],
        (NPAD, 128))

    xc0, xc1, xc2 = pl.kernel(
        _emb_body,
        out_type=[jax.ShapeDtypeStruct((NPAD, CW), f32)] * 3,
        mesh=mesh,
        scratch_types=[pltpu.VMEM((IDNB, IDB), jnp.int32)] * 2
        + [pltpu.VMEM((IDB, CW), f32)] * 3
        + [pltpu.SemaphoreType.DMA] * 3,
        compiler_params=pltpu.CompilerParams(use_tc_tiling_on_sc=False),
    )(nt_ids3, id_ids3, nt_a, nt_b, ident_table)

    def sc_aggregate(t0, t1):
        return pl.kernel(
            _agg_body,
            out_type=jax.ShapeDtypeStruct((NCORE, 2, NPAD, CW), f32),
            mesh=mesh,
            scratch_types=[pltpu.VMEM((2, SEGB, BLK), jnp.int32)]
            + [pltpu.VMEM((BLK, CW), f32)] * RING
            + [pltpu.VMEM((ZROWS, CW), f32),
               pltpu.VMEM_SHARED((NPAD, CW), f32)]
            + [pltpu.SemaphoreType.DMA] * (2 * RING),
            compiler_params=pltpu.CompilerParams(use_tc_tiling_on_sc=False),
        )(ed4, t0, t1)

    b32 = pl.BlockSpec((BR, CW), lambda i: (i, 0))
    b128 = pl.BlockSpec((BR, HID), lambda i: (i, 0))
    bagg = pl.BlockSpec((NCORE, 2, BR, CW), lambda i: (0, 0, i, 0))
    bw = pl.BlockSpec((HID, HID), lambda i: (0, 0))
    bb = pl.BlockSpec((1, HID), lambda i: (0, 0))

    def self_mm(chunks, w, b):
        return pl.pallas_call(
            _self_body,
            grid=(NBLOCKS,),
            in_specs=[b32] * 4 + [bw, bb],
            out_specs=b128,
            out_shape=jax.ShapeDtypeStruct((NPAD, HID), f32),
        )(*chunks, w, b)

    aggA0 = sc_aggregate(xc0, xc1)
    aggB0 = sc_aggregate(xc2, xc3)
    self0 = self_mm((xc0, xc1, xc2, xc3), w0p, bs0)
    x1c = pl.pallas_call(
        _l0_body,
        grid=(NBLOCKS,),
        in_specs=[b128, bagg, bagg, bw, bb],
        out_specs=[b32] * 4,
        out_shape=[jax.ShapeDtypeStruct((NPAD, CW), f32)] * 4,
    )(self0, aggA0, aggB0, wn0p, bn0)

    aggA1 = sc_aggregate(x1c[0], x1c[1])
    aggB1 = sc_aggregate(x1c[2], x1c[3])
    self1 = self_mm(x1c, W_self1, bs1)

    ba0 = pl.BlockSpec((NCORE, 1, BR, CW), lambda i: (0, 1, i, 0))
    bbi = pl.BlockSpec((BR, 128), lambda i: (i, 0))
    out = pl.pallas_call(
        _l1_body,
        grid=(NBLOCKS,),
        in_specs=[b128, bagg, bagg, ba0, bbi, bw, bb, bw, bb],
        out_specs=pl.BlockSpec((4 * 128, HID), lambda i: (0, 0)),
        out_shape=jax.ShapeDtypeStruct((4 * 128, HID), f32),
        scratch_shapes=[pltpu.VMEM((4 * 128, HID), f32)],
    )(self1, aggA1, aggB1, aggB0, bi_bc, W_neigh1, bn1, pw, pb)

    return (out[:G, 0], out[:G, 1:13])
